# baseline (speedup 1.0000x reference)
import jax
import jax.numpy as jnp
from jax.experimental import pallas as pl


def _copy_body(pe_ref, o_ref):
    o_ref[...] = pe_ref[...][None]


def kernel(x, pe):
    seq_len = x.shape[-1]
    d_model = pe.shape[-1]
    rows_per_block = 512
    grid = (seq_len // rows_per_block,)
    out = pl.pallas_call(
        _copy_body,
        grid=grid,
        in_specs=[pl.BlockSpec((rows_per_block, d_model), lambda i: (i, 0))],
        out_specs=pl.BlockSpec((1, rows_per_block, d_model), lambda i: (0, i, 0)),
        out_shape=jax.ShapeDtypeStruct((1, seq_len, d_model), pe.dtype),
    )(pe)
    return out
